# baseline (speedup 1.0000x reference)
import jax
import jax.numpy as jnp
from jax.experimental import pallas as pl
from jax.experimental.pallas import tpu as pltpu

_N = 4096
_B = 4
_K = 5
_F1 = 20
_F2 = 30
_C = 10
_CH = 128
_NCH = _N // _CH

_HP = jax.lax.Precision.DEFAULT
_BF = jnp.bfloat16


def _mega_kernel(lhbm_ref, x0_ref, m1_ref, m2_ref, b1_ref, b2_ref,
                 wfc_ref, msk_ref, sb_ref, sc_ref, bfc_ref, out_ref,
                 lb, f0, f1, hb, sa, sb_s, out2, sem0, sem1):
    p = pl.program_id(0)

    def lmul(full_f32):
        return jax.lax.dot_general(
            lb[...], full_f32.astype(_BF),
            dimension_numbers=(((1,), (0,)), ((), ())),
            preferred_element_type=jnp.float32, precision=_HP)

    m1 = m1_ref[...]
    m2 = m2_ref[...]

    @pl.when(p == 0)
    def _():
        def mkcopy(j, buf, sem):
            return pltpu.make_async_copy(
                lhbm_ref.at[pl.ds(j * _CH, _CH), :], buf, sem)

        mkcopy(0, f0, sem0).start()

        def body(j, carry):
            @pl.when(j + 1 < _NCH)
            def _():
                @pl.when(j % 2 == 0)
                def _():
                    mkcopy(j + 1, f1, sem1).start()

                @pl.when(j % 2 == 1)
                def _():
                    mkcopy(j + 1, f0, sem0).start()

            @pl.when(j % 2 == 0)
            def _():
                mkcopy(j, f0, sem0).wait()
                lb[pl.ds(j * _CH, _CH), :] = f0[...].astype(_BF)

            @pl.when(j % 2 == 1)
            def _():
                mkcopy(j, f1, sem1).wait()
                lb[pl.ds(j * _CH, _CH), :] = f1[...].astype(_BF)

            return carry

        jax.lax.fori_loop(0, _NCH, body, 0)

        t1 = lmul(x0_ref[...])
        hb[:, 0:4] = t1
        sa[:, 0:80] = (jax.lax.dot(x0_ref[...], m1[0:4], precision=_HP)
                       + jax.lax.dot(t1, m1[4:8], precision=_HP))

    @pl.when(p == 1)
    def _():
        t2 = 2.0 * lmul(hb[:, 0:4]) - x0_ref[...]
        hb[:, 4:8] = hb[:, 0:4]
        hb[:, 0:4] = t2
        sa[:, 0:80] += jax.lax.dot(t2, m1[8:12], precision=_HP)

    @pl.when(p == 2)
    def _():
        t3 = 2.0 * lmul(hb[:, 0:4]) - hb[:, 4:8]
        hb[:, 4:8] = hb[:, 0:4]
        hb[:, 0:4] = t3
        sa[:, 0:80] += jax.lax.dot(t3, m1[12:16], precision=_HP)

    @pl.when(p == 3)
    def _():
        t4 = 2.0 * lmul(hb[:, 0:4]) - hb[:, 4:8]
        acc = sa[:, 0:80] + jax.lax.dot(t4, m1[16:20], precision=_HP)
        hb[:, 0:80] = jnp.maximum(acc + b1_ref[...], 0.0)

    @pl.when(p == 4)
    def _():
        s1 = lmul(hb[:, 0:80])
        sa[:, 0:80] = s1
        out2[...] = (jax.lax.dot(hb[:, 0:80], m2[0:80], precision=_HP)
                        + jax.lax.dot(s1, m2[80:160], precision=_HP))

    @pl.when(p == 5)
    def _():
        s2 = 2.0 * lmul(sa[:, 0:80]) - hb[:, 0:80]
        sb_s[:, 0:80] = s2
        out2[...] += jax.lax.dot(s2, m2[160:240], precision=_HP)

    @pl.when(p == 6)
    def _():
        s3 = 2.0 * lmul(sb_s[:, 0:80]) - sa[:, 0:80]
        sa[:, 0:80] = s3
        out2[...] += jax.lax.dot(s3, m2[240:320], precision=_HP)

    @pl.when(p == 7)
    def _():
        s4 = 2.0 * lmul(sa[:, 0:80]) - sb_s[:, 0:80]
        acc = out2[...] + jax.lax.dot(s4, m2[320:400], precision=_HP)
        h2 = jnp.maximum(acc + b2_ref[...], 0.0)
        u = jax.lax.dot_general(h2.astype(_BF), wfc_ref[...],
                                dimension_numbers=(((0,), (0,)), ((), ())),
                                preferred_element_type=jnp.float32,
                                precision=_HP)
        um = u * msk_ref[...]
        logits = jax.lax.dot(
            sb_ref[...], jax.lax.dot(um, sc_ref[...], precision=_HP),
            precision=_HP) + bfc_ref[...]
        m = jnp.max(logits, axis=1, keepdims=True)
        z = logits - m
        lse = jnp.log(jnp.sum(jnp.exp(z), axis=1, keepdims=True))
        out_ref[...] = z - lse


def kernel(x, L, W1, b1, W2, b2, Wfc, bfc):
    B, N = _B, _N
    X0 = x[:, :, 0].T

    eyeB = jnp.eye(B, dtype=jnp.float32)
    M1 = (W1[:, 0, :][:, None, None, :] * eyeB[None, :, :, None]
          ).reshape(_K * B, B * _F1)
    M2 = (W2[:, None, :, None, :] * eyeB[None, :, None, :, None]
          ).reshape(_K * B * _F1, B * _F2)
    b1t = jnp.tile(b1, (B,))[None, :]
    b2t = jnp.tile(b2, (B,))[None, :]

    Wfcf = Wfc.reshape(_C, N, _F2).transpose(1, 0, 2).reshape(
        N, _C * _F2).astype(jnp.bfloat16)

    r = jnp.arange(B * _F2)[:, None]
    q = jnp.arange(_C * _F2)[None, :]
    msk = ((r % _F2) == (q % _F2)).astype(jnp.float32)
    sb = (jnp.arange(B)[:, None] == (jnp.arange(B * _F2)[None, :] // _F2)
          ).astype(jnp.float32)
    sc = ((jnp.arange(_C * _F2)[:, None] // _F2) == jnp.arange(_C)[None, :]
          ).astype(jnp.float32)
    bfcr = bfc[None, :]

    out = pl.pallas_call(
        _mega_kernel,
        grid=(2 * _K - 2,),
        in_specs=[
            pl.BlockSpec(memory_space=pltpu.HBM),
            pl.BlockSpec((_N, _B), lambda p: (0, 0)),
            pl.BlockSpec((_K * _B, _B * _F1), lambda p: (0, 0)),
            pl.BlockSpec((_K * _B * _F1, _B * _F2), lambda p: (0, 0)),
            pl.BlockSpec((1, _B * _F1), lambda p: (0, 0)),
            pl.BlockSpec((1, _B * _F2), lambda p: (0, 0)),
            pl.BlockSpec((_N, _C * _F2), lambda p: (0, 0)),
            pl.BlockSpec((_B * _F2, _C * _F2), lambda p: (0, 0)),
            pl.BlockSpec((_B, _B * _F2), lambda p: (0, 0)),
            pl.BlockSpec((_C * _F2, _C), lambda p: (0, 0)),
            pl.BlockSpec((1, _C), lambda p: (0, 0)),
        ],
        out_specs=pl.BlockSpec((_B, _C), lambda p: (0, 0)),
        out_shape=jax.ShapeDtypeStruct((B, _C), jnp.float32),
        scratch_shapes=[
            pltpu.VMEM((_N, _N), _BF),
            pltpu.VMEM((_CH, _N), jnp.float32),
            pltpu.VMEM((_CH, _N), jnp.float32),
            pltpu.VMEM((_N, 80), jnp.float32),
            pltpu.VMEM((_N, 80), jnp.float32),
            pltpu.VMEM((_N, 80), jnp.float32),
            pltpu.VMEM((_N, _B * _F2), jnp.float32),
            pltpu.SemaphoreType.DMA,
            pltpu.SemaphoreType.DMA,
        ],
        compiler_params=pltpu.CompilerParams(
            dimension_semantics=("arbitrary",),
            vmem_limit_bytes=100 * 1024 * 1024,
        ),
    )(L, X0, M1, M2, b1t, b2t, Wfcf, msk, sb, sc, bfcr)
    return out
